# Initial kernel scaffold; baseline (speedup 1.0000x reference)
import jax
import jax.numpy as jnp
from jax.experimental import pallas as pl


def kernel(x, edge_index, edge_attr, W1, b1, a1, W2, b2, a2, W3, b3, a3, W4, b4):
    raise NotImplementedError("write your pallas kernel here")


# speedup vs baseline: 2.0171x; 2.0171x over previous
import jax
import jax.numpy as jnp
from jax import lax
from jax.experimental import pallas as pl
from jax.experimental.pallas import tpu as pltpu
from jax.experimental.pallas import tpu_sc as plsc

N_NODES = 10000
N_EDGES = 160000
INDIM = 256
HIDDEN = 512
OUTDIM = 256
EDGEDIM = 16


NW = 32
EPW = N_EDGES // NW
GK = 40
GCH = EPW // GK


def _gather_body(x_hbm, row_hbm, col_hbm, sxx_hbm,
                 idx_r, idx_c, buf_r, buf_c, sem_r, sem_c):
    cid = lax.axis_index("c")
    sid = lax.axis_index("s")
    wid = sid * 2 + cid

    def chunk(ch, carry):
        base = wid * EPW + ch * GK
        pltpu.sync_copy(row_hbm.at[pl.ds(base, GK)], idx_r)
        pltpu.sync_copy(col_hbm.at[pl.ds(base, GK)], idx_c)
        cp_r = pltpu.async_copy(x_hbm.at[idx_r], buf_r, sem_r)
        cp_c = pltpu.async_copy(x_hbm.at[idx_c], buf_c, sem_c)
        cp_r.wait()
        cp_c.wait()
        pltpu.sync_copy(buf_r, sxx_hbm.at[pl.ds(base, GK), pl.ds(0, INDIM)])
        pltpu.sync_copy(buf_c, sxx_hbm.at[pl.ds(base, GK), pl.ds(INDIM, INDIM)])
        return carry

    lax.fori_loop(0, GCH, chunk, 0)


_sc_mesh = plsc.VectorSubcoreMesh(core_axis_name="c", subcore_axis_name="s")

_gather = pl.kernel(
    _gather_body,
    out_type=jax.ShapeDtypeStruct((N_EDGES, 2 * INDIM), jnp.float32),
    mesh=_sc_mesh,
    scratch_types=[
        pltpu.VMEM((GK,), jnp.int32),
        pltpu.VMEM((GK,), jnp.int32),
        pltpu.VMEM((GK, INDIM), jnp.float32),
        pltpu.VMEM((GK, INDIM), jnp.float32),
        pltpu.SemaphoreType.DMA,
        pltpu.SemaphoreType.DMA,
    ],
)


E_BLK = 1600
N_CC = 4
CW = HIDDEN // N_CC


def _edge_body(sxx_ref, ea_ref, w1ab_ref, w1c_ref, w2_ref,
               b1_ref, b2_ref, a1_ref, a2_ref, out_ref):
    z = jnp.dot(sxx_ref[...], w1ab_ref[...], preferred_element_type=jnp.float32)
    z = z + jnp.dot(ea_ref[...], w1c_ref[...], preferred_element_type=jnp.float32)
    z = z + b1_ref[...]
    a1 = a1_ref[0, 0]
    e1 = jnp.maximum(z, 0.0) + a1 * jnp.minimum(z, 0.0)
    z2 = jnp.dot(e1, w2_ref[...], preferred_element_type=jnp.float32) + b2_ref[...]
    a2 = a2_ref[0, 0]
    e2 = jnp.maximum(z2, 0.0) + a2 * jnp.minimum(z2, 0.0)
    for c in range(N_CC):
        out_ref[c] = e2[:, c * CW:(c + 1) * CW]


def _edge_mlp(sxx, ea, w1ab, w1c, w2, b1, b2, a1, a2):
    grid = (N_EDGES // E_BLK,)
    return pl.pallas_call(
        _edge_body,
        grid=grid,
        in_specs=[
            pl.BlockSpec((E_BLK, 2 * INDIM), lambda i: (i, 0)),
            pl.BlockSpec((E_BLK, EDGEDIM), lambda i: (i, 0)),
            pl.BlockSpec((2 * INDIM, HIDDEN), lambda i: (0, 0)),
            pl.BlockSpec((EDGEDIM, HIDDEN), lambda i: (0, 0)),
            pl.BlockSpec((HIDDEN, HIDDEN), lambda i: (0, 0)),
            pl.BlockSpec((1, HIDDEN), lambda i: (0, 0)),
            pl.BlockSpec((1, HIDDEN), lambda i: (0, 0)),
            pl.BlockSpec((1, 1), lambda i: (0, 0)),
            pl.BlockSpec((1, 1), lambda i: (0, 0)),
        ],
        out_specs=pl.BlockSpec((N_CC, E_BLK, CW), lambda i: (0, i, 0)),
        out_shape=jax.ShapeDtypeStruct((N_CC, N_EDGES, CW), jnp.float32),
    )(sxx, ea, w1ab, w1c, w2, b1, b2, a1, a2)


N_TILES = 16
ET = N_EDGES // N_TILES
SK = 80
SCH = ET // SK
RPT = 624
TAIL = N_NODES - N_TILES * RPT
ZR = 208


def _scatter_body(e2_hbm, row_hbm, agg_hbm, idx, ebuf, zbuf, shared):
    cid = lax.axis_index("c")
    sid = lax.axis_index("s")

    def zloop(t, carry):
        i = t // 8
        j = t - i * 8
        zbuf[i, pl.ds(j * 16, 16)] = jnp.zeros((16,), jnp.float32)
        return carry

    lax.fori_loop(0, ZR * 8, zloop, 0)

    for cc in range(2):
        c_idx = cid * 2 + cc

        for k in range(3):
            pltpu.sync_copy(zbuf, shared.at[pl.ds(sid * RPT + k * ZR, ZR)])

        @pl.when(sid == N_TILES - 1)
        def _zero_tail():
            pltpu.sync_copy(zbuf.at[pl.ds(0, TAIL)],
                            shared.at[pl.ds(N_TILES * RPT, TAIL)])

        plsc.subcore_barrier()

        def chunk(ch, carry):
            base = sid * ET + ch * SK
            pltpu.sync_copy(row_hbm.at[pl.ds(base, SK)], idx)
            pltpu.sync_copy(e2_hbm.at[c_idx, pl.ds(base, SK)], ebuf)
            pltpu.sync_copy(ebuf, shared.at[idx], add=True)
            return carry

        lax.fori_loop(0, SCH, chunk, 0)
        plsc.subcore_barrier()

        pltpu.sync_copy(shared.at[pl.ds(sid * RPT, RPT)],
                        agg_hbm.at[c_idx, pl.ds(sid * RPT, RPT)])

        @pl.when(sid == N_TILES - 1)
        def _write_tail():
            pltpu.sync_copy(shared.at[pl.ds(N_TILES * RPT, TAIL)],
                            agg_hbm.at[c_idx, pl.ds(N_TILES * RPT, TAIL)])

        plsc.subcore_barrier()


_scatter = pl.kernel(
    _scatter_body,
    out_type=jax.ShapeDtypeStruct((N_CC, N_NODES, CW), jnp.float32),
    mesh=_sc_mesh,
    scratch_types=[
        pltpu.VMEM((SK,), jnp.int32),
        pltpu.VMEM((SK, CW), jnp.float32),
        pltpu.VMEM((ZR, CW), jnp.float32),
        pltpu.VMEM_SHARED((N_NODES, CW), jnp.float32),
    ],
)


V_BLK = 1000


def _node_body(x_ref, aggr_ref, w3a_ref, w3b_ref, w4_ref,
               b3_ref, b4_ref, a3_ref, out_ref):
    z = jnp.dot(x_ref[...], w3a_ref[...], preferred_element_type=jnp.float32)
    w3b = w3b_ref[...]
    for c in range(N_CC):
        z = z + jnp.dot(aggr_ref[c], w3b[c * CW:(c + 1) * CW, :],
                        preferred_element_type=jnp.float32)
    z = z + b3_ref[...]
    a3 = a3_ref[0, 0]
    h = jnp.maximum(z, 0.0) + a3 * jnp.minimum(z, 0.0)
    out_ref[...] = jnp.dot(h, w4_ref[...], preferred_element_type=jnp.float32) \
        + b4_ref[...]


def _node_mlp(x, aggr, w3a, w3b, w4, b3, b4, a3):
    grid = (N_NODES // V_BLK,)
    return pl.pallas_call(
        _node_body,
        grid=grid,
        in_specs=[
            pl.BlockSpec((V_BLK, INDIM), lambda i: (i, 0)),
            pl.BlockSpec((N_CC, V_BLK, CW), lambda i: (0, i, 0)),
            pl.BlockSpec((INDIM, HIDDEN), lambda i: (0, 0)),
            pl.BlockSpec((HIDDEN, HIDDEN), lambda i: (0, 0)),
            pl.BlockSpec((HIDDEN, OUTDIM), lambda i: (0, 0)),
            pl.BlockSpec((1, HIDDEN), lambda i: (0, 0)),
            pl.BlockSpec((1, OUTDIM), lambda i: (0, 0)),
            pl.BlockSpec((1, 1), lambda i: (0, 0)),
        ],
        out_specs=pl.BlockSpec((V_BLK, OUTDIM), lambda i: (i, 0)),
        out_shape=jax.ShapeDtypeStruct((N_NODES, OUTDIM), jnp.float32),
    )(x, aggr, w3a, w3b, w4, b3, b4, a3)


def kernel(x, edge_index, edge_attr, W1, b1, a1, W2, b2, a2, W3, b3, a3, W4, b4):
    row = edge_index[0].astype(jnp.int32)
    col = edge_index[1].astype(jnp.int32)
    w1ab = W1[:, :2 * INDIM].T
    w1c = W1[:, 2 * INDIM:].T
    w2 = W2.T
    w3a = W3[:, :INDIM].T
    w3b = W3[:, INDIM:].T
    w4 = W4.T
    b1r = b1.reshape(1, HIDDEN)
    b2r = b2.reshape(1, HIDDEN)
    b3r = b3.reshape(1, HIDDEN)
    b4r = b4.reshape(1, OUTDIM)
    a1r = jnp.reshape(a1, (1, 1))
    a2r = jnp.reshape(a2, (1, 1))
    a3r = jnp.reshape(a3, (1, 1))

    sxx = _gather(x, row, col)
    e2 = _edge_mlp(sxx, edge_attr, w1ab, w1c, w2, b1r, b2r, a1r, a2r)
    aggr = _scatter(e2, row)
    out = _node_mlp(x, aggr, w3a, w3b, w4, b3r, b4r, a3r)
    return out


# speedup vs baseline: 3.0805x; 1.5272x over previous
import jax
import jax.numpy as jnp
from jax import lax
from jax.experimental import pallas as pl
from jax.experimental.pallas import tpu as pltpu
from jax.experimental.pallas import tpu_sc as plsc

N_NODES = 10000
N_EDGES = 160000
INDIM = 256
HIDDEN = 512
OUTDIM = 256
EDGEDIM = 16


NW = 32
EPW = N_EDGES // NW
GK = 40
GCH = EPW // GK


NBUF = 4


def _gather_body(x_hbm, row_hbm, col_hbm, sxx_hbm,
                 idxr, idxc,
                 bufr0, bufr1, bufr2, bufr3,
                 bufc0, bufc1, bufc2, bufc3,
                 gsr0, gsr1, gsr2, gsr3,
                 gsc0, gsc1, gsc2, gsc3,
                 wsr0, wsr1, wsr2, wsr3,
                 wsc0, wsc1, wsc2, wsc3):
    cid = lax.axis_index("c")
    sid = lax.axis_index("s")
    wid = sid * 2 + cid
    ebase = wid * EPW
    bufr = (bufr0, bufr1, bufr2, bufr3)
    bufc = (bufc0, bufc1, bufc2, bufc3)
    gsr = (gsr0, gsr1, gsr2, gsr3)
    gsc = (gsc0, gsc1, gsc2, gsc3)
    wsr = (wsr0, wsr1, wsr2, wsr3)
    wsc = (wsc0, wsc1, wsc2, wsc3)

    pltpu.sync_copy(row_hbm.at[pl.ds(ebase, EPW)], idxr)
    pltpu.sync_copy(col_hbm.at[pl.ds(ebase, EPW)], idxc)

    def issue_gather(ch, b):
        s = pl.ds(ch * GK, GK)
        pltpu.async_copy(x_hbm.at[idxr.at[s]], bufr[b], gsr[b])
        pltpu.async_copy(x_hbm.at[idxc.at[s]], bufc[b], gsc[b])

    def wait_gather(b):
        pltpu.make_async_copy(x_hbm.at[idxr.at[pl.ds(0, GK)]], bufr[b], gsr[b]).wait()
        pltpu.make_async_copy(x_hbm.at[idxc.at[pl.ds(0, GK)]], bufc[b], gsc[b]).wait()

    def issue_writes(ch, b):
        base = ebase + ch * GK
        pltpu.async_copy(bufr[b], sxx_hbm.at[pl.ds(base, GK), pl.ds(0, INDIM)], wsr[b])
        pltpu.async_copy(bufc[b], sxx_hbm.at[pl.ds(base, GK), pl.ds(INDIM, INDIM)], wsc[b])

    def drain_writes(b):
        pltpu.make_async_copy(bufr[b], sxx_hbm.at[pl.ds(ebase, GK), pl.ds(0, INDIM)], wsr[b]).wait()
        pltpu.make_async_copy(bufc[b], sxx_hbm.at[pl.ds(ebase, GK), pl.ds(INDIM, INDIM)], wsc[b]).wait()

    for b in range(NBUF):
        issue_gather(b, b)

    def body(g, carry):
        for b in range(NBUF):
            ch = NBUF * g + b

            @pl.when(ch < GCH)
            def _():
                wait_gather(b)
                issue_writes(ch, b)
        for b in range(NBUF):
            chn = NBUF * g + b + NBUF

            @pl.when(chn < GCH)
            def _():
                drain_writes(b)
                issue_gather(chn, b)
        return carry

    lax.fori_loop(0, (GCH + NBUF - 1) // NBUF, body, 0)
    for b in range(NBUF):
        drain_writes(b)


_sc_mesh = plsc.VectorSubcoreMesh(core_axis_name="c", subcore_axis_name="s")

_gather = pl.kernel(
    _gather_body,
    out_type=jax.ShapeDtypeStruct((N_EDGES, 2 * INDIM), jnp.float32),
    mesh=_sc_mesh,
    scratch_types=(
        [pltpu.VMEM((EPW,), jnp.int32)] * 2
        + [pltpu.VMEM((GK, INDIM), jnp.float32)] * (2 * NBUF)
        + [pltpu.SemaphoreType.DMA] * (4 * NBUF)
    ),
)


E_BLK = 1600
N_CC = 4
CW = HIDDEN // N_CC


def _edge_body(sxx_ref, ea_ref, w1ab_ref, w1c_ref, w2_ref,
               b1_ref, b2_ref, a1_ref, a2_ref, out_ref):
    z = jnp.dot(sxx_ref[...], w1ab_ref[...], preferred_element_type=jnp.float32)
    z = z + jnp.dot(ea_ref[...], w1c_ref[...], preferred_element_type=jnp.float32)
    z = z + b1_ref[...]
    a1 = a1_ref[0, 0]
    e1 = jnp.maximum(z, 0.0) + a1 * jnp.minimum(z, 0.0)
    z2 = jnp.dot(e1, w2_ref[...], preferred_element_type=jnp.float32) + b2_ref[...]
    a2 = a2_ref[0, 0]
    e2 = jnp.maximum(z2, 0.0) + a2 * jnp.minimum(z2, 0.0)
    for c in range(N_CC):
        out_ref[c] = e2[:, c * CW:(c + 1) * CW]


def _edge_mlp(sxx, ea, w1ab, w1c, w2, b1, b2, a1, a2):
    grid = (N_EDGES // E_BLK,)
    return pl.pallas_call(
        _edge_body,
        grid=grid,
        in_specs=[
            pl.BlockSpec((E_BLK, 2 * INDIM), lambda i: (i, 0)),
            pl.BlockSpec((E_BLK, EDGEDIM), lambda i: (i, 0)),
            pl.BlockSpec((2 * INDIM, HIDDEN), lambda i: (0, 0)),
            pl.BlockSpec((EDGEDIM, HIDDEN), lambda i: (0, 0)),
            pl.BlockSpec((HIDDEN, HIDDEN), lambda i: (0, 0)),
            pl.BlockSpec((1, HIDDEN), lambda i: (0, 0)),
            pl.BlockSpec((1, HIDDEN), lambda i: (0, 0)),
            pl.BlockSpec((1, 1), lambda i: (0, 0)),
            pl.BlockSpec((1, 1), lambda i: (0, 0)),
        ],
        out_specs=pl.BlockSpec((N_CC, E_BLK, CW), lambda i: (0, i, 0)),
        out_shape=jax.ShapeDtypeStruct((N_CC, N_EDGES, CW), jnp.float32),
    )(sxx, ea, w1ab, w1c, w2, b1, b2, a1, a2)


N_TILES = 16
ET = N_EDGES // N_TILES
SK = 80
SCH = ET // SK
RPT = 624
TAIL = N_NODES - N_TILES * RPT
ZR = 104


def _scatter_body(e2_hbm, row2d_hbm, agg_hbm,
                  idx2d, ebuf0, ebuf1, zbuf, shared, es0, es1):
    cid = lax.axis_index("c")
    sid = lax.axis_index("s")
    ebuf = (ebuf0, ebuf1)
    es = (es0, es1)

    pltpu.sync_copy(row2d_hbm.at[sid], idx2d)

    def zloop(t, carry):
        i = t // 8
        j = t - i * 8
        zbuf[i, pl.ds(j * 16, 16)] = jnp.zeros((16,), jnp.float32)
        return carry

    lax.fori_loop(0, ZR * 8, zloop, 0)

    for cc in range(2):
        c_idx = cid * 2 + cc

        for k in range(6):
            pltpu.sync_copy(zbuf, shared.at[pl.ds(sid * RPT + k * ZR, ZR)])

        @pl.when(sid == N_TILES - 1)
        def _zero_tail():
            pltpu.sync_copy(zbuf.at[pl.ds(0, TAIL)],
                            shared.at[pl.ds(N_TILES * RPT, TAIL)])

        plsc.subcore_barrier()

        def issue_read(ch, b):
            pltpu.async_copy(e2_hbm.at[c_idx, pl.ds(sid * ET + ch * SK, SK)],
                             ebuf[b], es[b])

        def wait_read(b):
            pltpu.make_async_copy(e2_hbm.at[c_idx, pl.ds(sid * ET, SK)],
                                  ebuf[b], es[b]).wait()

        issue_read(0, 0)
        issue_read(1, 1)

        def chunk(g, carry):
            for b in range(2):
                ch = 2 * g + b

                @pl.when(ch < SCH)
                def _():
                    wait_read(b)
                    pltpu.sync_copy(ebuf[b], shared.at[idx2d.at[ch]], add=True)

                    @pl.when(ch + 2 < SCH)
                    def _():
                        issue_read(ch + 2, b)
            return carry

        lax.fori_loop(0, (SCH + 1) // 2, chunk, 0)
        plsc.subcore_barrier()

        pltpu.sync_copy(shared.at[pl.ds(sid * RPT, RPT)],
                        agg_hbm.at[c_idx, pl.ds(sid * RPT, RPT)])

        @pl.when(sid == N_TILES - 1)
        def _write_tail():
            pltpu.sync_copy(shared.at[pl.ds(N_TILES * RPT, TAIL)],
                            agg_hbm.at[c_idx, pl.ds(N_TILES * RPT, TAIL)])

        plsc.subcore_barrier()


_scatter = pl.kernel(
    _scatter_body,
    out_type=jax.ShapeDtypeStruct((N_CC, N_NODES, CW), jnp.float32),
    mesh=_sc_mesh,
    scratch_types=[
        pltpu.VMEM((SCH, SK), jnp.int32),
        pltpu.VMEM((SK, CW), jnp.float32),
        pltpu.VMEM((SK, CW), jnp.float32),
        pltpu.VMEM((ZR, CW), jnp.float32),
        pltpu.VMEM_SHARED((N_NODES, CW), jnp.float32),
        pltpu.SemaphoreType.DMA,
        pltpu.SemaphoreType.DMA,
    ],
)


V_BLK = 1000


def _node_body(x_ref, aggr_ref, w3a_ref, w3b_ref, w4_ref,
               b3_ref, b4_ref, a3_ref, out_ref):
    z = jnp.dot(x_ref[...], w3a_ref[...], preferred_element_type=jnp.float32)
    w3b = w3b_ref[...]
    for c in range(N_CC):
        z = z + jnp.dot(aggr_ref[c], w3b[c * CW:(c + 1) * CW, :],
                        preferred_element_type=jnp.float32)
    z = z + b3_ref[...]
    a3 = a3_ref[0, 0]
    h = jnp.maximum(z, 0.0) + a3 * jnp.minimum(z, 0.0)
    out_ref[...] = jnp.dot(h, w4_ref[...], preferred_element_type=jnp.float32) \
        + b4_ref[...]


def _node_mlp(x, aggr, w3a, w3b, w4, b3, b4, a3):
    grid = (N_NODES // V_BLK,)
    return pl.pallas_call(
        _node_body,
        grid=grid,
        in_specs=[
            pl.BlockSpec((V_BLK, INDIM), lambda i: (i, 0)),
            pl.BlockSpec((N_CC, V_BLK, CW), lambda i: (0, i, 0)),
            pl.BlockSpec((INDIM, HIDDEN), lambda i: (0, 0)),
            pl.BlockSpec((HIDDEN, HIDDEN), lambda i: (0, 0)),
            pl.BlockSpec((HIDDEN, OUTDIM), lambda i: (0, 0)),
            pl.BlockSpec((1, HIDDEN), lambda i: (0, 0)),
            pl.BlockSpec((1, OUTDIM), lambda i: (0, 0)),
            pl.BlockSpec((1, 1), lambda i: (0, 0)),
        ],
        out_specs=pl.BlockSpec((V_BLK, OUTDIM), lambda i: (i, 0)),
        out_shape=jax.ShapeDtypeStruct((N_NODES, OUTDIM), jnp.float32),
    )(x, aggr, w3a, w3b, w4, b3, b4, a3)


def kernel(x, edge_index, edge_attr, W1, b1, a1, W2, b2, a2, W3, b3, a3, W4, b4):
    row = edge_index[0].astype(jnp.int32)
    col = edge_index[1].astype(jnp.int32)
    w1ab = W1[:, :2 * INDIM].T
    w1c = W1[:, 2 * INDIM:].T
    w2 = W2.T
    w3a = W3[:, :INDIM].T
    w3b = W3[:, INDIM:].T
    w4 = W4.T
    b1r = b1.reshape(1, HIDDEN)
    b2r = b2.reshape(1, HIDDEN)
    b3r = b3.reshape(1, HIDDEN)
    b4r = b4.reshape(1, OUTDIM)
    a1r = jnp.reshape(a1, (1, 1))
    a2r = jnp.reshape(a2, (1, 1))
    a3r = jnp.reshape(a3, (1, 1))

    row2d = row.reshape(N_TILES, SCH, SK)
    sxx = _gather(x, row, col)
    e2 = _edge_mlp(sxx, edge_attr, w1ab, w1c, w2, b1r, b2r, a1r, a2r)
    aggr = _scatter(e2, row2d)
    out = _node_mlp(x, aggr, w3a, w3b, w4, b3r, b4r, a3r)
    return out


# speedup vs baseline: 3.4321x; 1.1141x over previous
import jax
import jax.numpy as jnp
from jax import lax
from jax.experimental import pallas as pl
from jax.experimental.pallas import tpu as pltpu
from jax.experimental.pallas import tpu_sc as plsc

N_NODES = 10000
N_EDGES = 160000
INDIM = 256
HIDDEN = 512
OUTDIM = 256
EDGEDIM = 16

NW = 32
GK = 40
NBUF = 4
N_CC = 4
CW = HIDDEN // N_CC
N_TILES = 16
SK = 80
RPT = 624
TAIL = N_NODES - N_TILES * RPT
ZR = 104
E_BLK = 1280

GROUPS = ((0, 80640), (80640, 79360))

_sc_mesh = plsc.VectorSubcoreMesh(core_axis_name="c", subcore_axis_name="s")


def _make_gather(ng):
    epw = ng // NW
    gch = epw // GK

    def body(x_hbm, row_hbm, col_hbm, sxx_hbm,
             idxr, idxc,
             bufr0, bufr1, bufr2, bufr3,
             bufc0, bufc1, bufc2, bufc3,
             gsr0, gsr1, gsr2, gsr3,
             gsc0, gsc1, gsc2, gsc3,
             wsr0, wsr1, wsr2, wsr3,
             wsc0, wsc1, wsc2, wsc3):
        cid = lax.axis_index("c")
        sid = lax.axis_index("s")
        wid = sid * 2 + cid
        ebase = wid * epw
        bufr = (bufr0, bufr1, bufr2, bufr3)
        bufc = (bufc0, bufc1, bufc2, bufc3)
        gsr = (gsr0, gsr1, gsr2, gsr3)
        gsc = (gsc0, gsc1, gsc2, gsc3)
        wsr = (wsr0, wsr1, wsr2, wsr3)
        wsc = (wsc0, wsc1, wsc2, wsc3)

        pltpu.sync_copy(row_hbm.at[pl.ds(ebase, epw)], idxr)
        pltpu.sync_copy(col_hbm.at[pl.ds(ebase, epw)], idxc)

        def issue_gather(ch, b):
            s = pl.ds(ch * GK, GK)
            pltpu.async_copy(x_hbm.at[idxr.at[s]], bufr[b], gsr[b])
            pltpu.async_copy(x_hbm.at[idxc.at[s]], bufc[b], gsc[b])

        def wait_gather(b):
            pltpu.make_async_copy(x_hbm.at[idxr.at[pl.ds(0, GK)]], bufr[b], gsr[b]).wait()
            pltpu.make_async_copy(x_hbm.at[idxc.at[pl.ds(0, GK)]], bufc[b], gsc[b]).wait()

        def issue_writes(ch, b):
            base = ebase + ch * GK
            pltpu.async_copy(bufr[b], sxx_hbm.at[pl.ds(base, GK), pl.ds(0, INDIM)], wsr[b])
            pltpu.async_copy(bufc[b], sxx_hbm.at[pl.ds(base, GK), pl.ds(INDIM, INDIM)], wsc[b])

        def drain_writes(b):
            pltpu.make_async_copy(bufr[b], sxx_hbm.at[pl.ds(ebase, GK), pl.ds(0, INDIM)], wsr[b]).wait()
            pltpu.make_async_copy(bufc[b], sxx_hbm.at[pl.ds(ebase, GK), pl.ds(INDIM, INDIM)], wsc[b]).wait()

        for b in range(NBUF):
            issue_gather(b, b)

        def loop(g, carry):
            for b in range(NBUF):
                ch = NBUF * g + b

                @pl.when(ch < gch)
                def _():
                    wait_gather(b)
                    issue_writes(ch, b)
            for b in range(NBUF):
                chn = NBUF * g + b + NBUF

                @pl.when(chn < gch)
                def _():
                    drain_writes(b)
                    issue_gather(chn, b)
            return carry

        lax.fori_loop(0, (gch + NBUF - 1) // NBUF, loop, 0)
        for b in range(NBUF):
            drain_writes(b)

    return pl.kernel(
        body,
        out_type=jax.ShapeDtypeStruct((ng, 2 * INDIM), jnp.float32),
        mesh=_sc_mesh,
        scratch_types=(
            [pltpu.VMEM((epw,), jnp.int32)] * 2
            + [pltpu.VMEM((GK, INDIM), jnp.float32)] * (2 * NBUF)
            + [pltpu.SemaphoreType.DMA] * (4 * NBUF)
        ),
    )


_gathers = tuple(_make_gather(ng) for _, ng in GROUPS)


def _edge_body(sxx_ref, ea_ref, w1ab_ref, w1c_ref, w2_ref,
               b1_ref, b2_ref, a1_ref, a2_ref, out_ref):
    z = jnp.dot(sxx_ref[...], w1ab_ref[...], preferred_element_type=jnp.float32)
    z = z + jnp.dot(ea_ref[...], w1c_ref[...], preferred_element_type=jnp.float32)
    z = z + b1_ref[...]
    a1 = a1_ref[0, 0]
    e1 = jnp.maximum(z, 0.0) + a1 * jnp.minimum(z, 0.0)
    z2 = jnp.dot(e1, w2_ref[...], preferred_element_type=jnp.float32) + b2_ref[...]
    a2 = a2_ref[0, 0]
    e2 = jnp.maximum(z2, 0.0) + a2 * jnp.minimum(z2, 0.0)
    for c in range(N_CC):
        out_ref[c] = e2[:, c * CW:(c + 1) * CW]


def _edge_mlp(sxx, ea, w1ab, w1c, w2, b1, b2, a1, a2):
    ng = sxx.shape[0]
    return pl.pallas_call(
        _edge_body,
        grid=(ng // E_BLK,),
        in_specs=[
            pl.BlockSpec((E_BLK, 2 * INDIM), lambda i: (i, 0)),
            pl.BlockSpec((E_BLK, EDGEDIM), lambda i: (i, 0)),
            pl.BlockSpec((2 * INDIM, HIDDEN), lambda i: (0, 0)),
            pl.BlockSpec((EDGEDIM, HIDDEN), lambda i: (0, 0)),
            pl.BlockSpec((HIDDEN, HIDDEN), lambda i: (0, 0)),
            pl.BlockSpec((1, HIDDEN), lambda i: (0, 0)),
            pl.BlockSpec((1, HIDDEN), lambda i: (0, 0)),
            pl.BlockSpec((1, 1), lambda i: (0, 0)),
            pl.BlockSpec((1, 1), lambda i: (0, 0)),
        ],
        out_specs=pl.BlockSpec((N_CC, E_BLK, CW), lambda i: (0, i, 0)),
        out_shape=jax.ShapeDtypeStruct((N_CC, ng, CW), jnp.float32),
    )(sxx, ea, w1ab, w1c, w2, b1, b2, a1, a2)


def _make_scatter(ng):
    et = ng // N_TILES
    sch = et // SK

    def body(e2_hbm, row3d_hbm, agg_hbm,
             idx2d, ebuf0, ebuf1, zbuf, shared, es0, es1):
        cid = lax.axis_index("c")
        sid = lax.axis_index("s")
        ebuf = (ebuf0, ebuf1)
        es = (es0, es1)

        pltpu.sync_copy(row3d_hbm.at[sid], idx2d)

        def zloop(t, carry):
            i = t // 8
            j = t - i * 8
            zbuf[i, pl.ds(j * 16, 16)] = jnp.zeros((16,), jnp.float32)
            return carry

        lax.fori_loop(0, ZR * 8, zloop, 0)

        for cc in range(2):
            c_idx = cid * 2 + cc

            for k in range(6):
                pltpu.sync_copy(zbuf, shared.at[pl.ds(sid * RPT + k * ZR, ZR)])

            @pl.when(sid == N_TILES - 1)
            def _zero_tail():
                pltpu.sync_copy(zbuf.at[pl.ds(0, TAIL)],
                                shared.at[pl.ds(N_TILES * RPT, TAIL)])

            plsc.subcore_barrier()

            def issue_read(ch, b):
                pltpu.async_copy(e2_hbm.at[c_idx, pl.ds(sid * et + ch * SK, SK)],
                                 ebuf[b], es[b])

            def wait_read(b):
                pltpu.make_async_copy(e2_hbm.at[c_idx, pl.ds(sid * et, SK)],
                                      ebuf[b], es[b]).wait()

            issue_read(0, 0)
            issue_read(1, 1)

            def chunk(g, carry):
                for b in range(2):
                    ch = 2 * g + b

                    @pl.when(ch < sch)
                    def _():
                        wait_read(b)
                        pltpu.sync_copy(ebuf[b], shared.at[idx2d.at[ch]], add=True)

                        @pl.when(ch + 2 < sch)
                        def _():
                            issue_read(ch + 2, b)
                return carry

            lax.fori_loop(0, (sch + 1) // 2, chunk, 0)
            plsc.subcore_barrier()

            pltpu.sync_copy(shared.at[pl.ds(sid * RPT, RPT)],
                            agg_hbm.at[c_idx, pl.ds(sid * RPT, RPT)])

            @pl.when(sid == N_TILES - 1)
            def _write_tail():
                pltpu.sync_copy(shared.at[pl.ds(N_TILES * RPT, TAIL)],
                                agg_hbm.at[c_idx, pl.ds(N_TILES * RPT, TAIL)])

            plsc.subcore_barrier()

    return pl.kernel(
        body,
        out_type=jax.ShapeDtypeStruct((N_CC, N_NODES, CW), jnp.float32),
        mesh=_sc_mesh,
        scratch_types=[
            pltpu.VMEM((sch, SK), jnp.int32),
            pltpu.VMEM((SK, CW), jnp.float32),
            pltpu.VMEM((SK, CW), jnp.float32),
            pltpu.VMEM((ZR, CW), jnp.float32),
            pltpu.VMEM_SHARED((N_NODES, CW), jnp.float32),
            pltpu.SemaphoreType.DMA,
            pltpu.SemaphoreType.DMA,
        ],
    )


_scatters = tuple(_make_scatter(ng) for _, ng in GROUPS)


V_BLK = 1000


def _node_body(x_ref, aggr0_ref, aggr1_ref, w3a_ref, w3b_ref, w4_ref,
               b3_ref, b4_ref, a3_ref, out_ref):
    z = jnp.dot(x_ref[...], w3a_ref[...], preferred_element_type=jnp.float32)
    w3b = w3b_ref[...]
    for c in range(N_CC):
        agg_c = aggr0_ref[c] + aggr1_ref[c]
        z = z + jnp.dot(agg_c, w3b[c * CW:(c + 1) * CW, :],
                        preferred_element_type=jnp.float32)
    z = z + b3_ref[...]
    a3 = a3_ref[0, 0]
    h = jnp.maximum(z, 0.0) + a3 * jnp.minimum(z, 0.0)
    out_ref[...] = jnp.dot(h, w4_ref[...], preferred_element_type=jnp.float32) \
        + b4_ref[...]


def _node_mlp(x, aggr0, aggr1, w3a, w3b, w4, b3, b4, a3):
    agg_spec = pl.BlockSpec((N_CC, V_BLK, CW), lambda i: (0, i, 0))
    return pl.pallas_call(
        _node_body,
        grid=(N_NODES // V_BLK,),
        in_specs=[
            pl.BlockSpec((V_BLK, INDIM), lambda i: (i, 0)),
            agg_spec,
            agg_spec,
            pl.BlockSpec((INDIM, HIDDEN), lambda i: (0, 0)),
            pl.BlockSpec((HIDDEN, HIDDEN), lambda i: (0, 0)),
            pl.BlockSpec((HIDDEN, OUTDIM), lambda i: (0, 0)),
            pl.BlockSpec((1, HIDDEN), lambda i: (0, 0)),
            pl.BlockSpec((1, OUTDIM), lambda i: (0, 0)),
            pl.BlockSpec((1, 1), lambda i: (0, 0)),
        ],
        out_specs=pl.BlockSpec((V_BLK, OUTDIM), lambda i: (i, 0)),
        out_shape=jax.ShapeDtypeStruct((N_NODES, OUTDIM), jnp.float32),
    )(x, aggr0, aggr1, w3a, w3b, w4, b3, b4, a3)


def kernel(x, edge_index, edge_attr, W1, b1, a1, W2, b2, a2, W3, b3, a3, W4, b4):
    row = edge_index[0].astype(jnp.int32)
    col = edge_index[1].astype(jnp.int32)
    w1ab = W1[:, :2 * INDIM].T
    w1c = W1[:, 2 * INDIM:].T
    w2 = W2.T
    w3a = W3[:, :INDIM].T
    w3b = W3[:, INDIM:].T
    w4 = W4.T
    b1r = b1.reshape(1, HIDDEN)
    b2r = b2.reshape(1, HIDDEN)
    b3r = b3.reshape(1, HIDDEN)
    b4r = b4.reshape(1, OUTDIM)
    a1r = jnp.reshape(a1, (1, 1))
    a2r = jnp.reshape(a2, (1, 1))
    a3r = jnp.reshape(a3, (1, 1))

    aggs = []
    for gi, (start, ng) in enumerate(GROUPS):
        row_g = lax.dynamic_slice_in_dim(row, start, ng)
        col_g = lax.dynamic_slice_in_dim(col, start, ng)
        ea_g = lax.dynamic_slice_in_dim(edge_attr, start, ng)
        sxx = _gathers[gi](x, row_g, col_g)
        e2 = _edge_mlp(sxx, ea_g, w1ab, w1c, w2, b1r, b2r, a1r, a2r)
        row3d = row_g.reshape(N_TILES, ng // N_TILES // SK, SK)
        aggs.append(_scatters[gi](e2, row3d))

    return _node_mlp(x, aggs[0], aggs[1], w3a, w3b, w4, b3r, b4r, a3r)
